# baseline (speedup 1.0000x reference)
import functools

import jax
import jax.numpy as jnp
from jax import lax
from jax.experimental import pallas as pl
from jax.experimental.pallas import tpu as pltpu
from jax.experimental.pallas import tpu_sc as plsc

S = 2048
H = 768
INNER = 768
NE = 64
TM = 8
TILES = 320
NROWS = TILES * TM
BLK = 128

_NC, _NS = 2, 16
NW = _NC * _NS
TOK_W = S // NW


def _route_body(x_ref, wrt_ref, dest_ref, lo_ref, nt_ref):
    x = x_ref[...]
    logits = jnp.dot(x, wrt_ref[...], preferred_element_type=jnp.float32)
    iota_e = lax.broadcasted_iota(jnp.int32, (S, NE), 1)
    mx = jnp.max(logits, axis=1, keepdims=True)
    eid = jnp.min(jnp.where(logits == mx, iota_e, NE), axis=1, keepdims=True)
    onehot = (iota_e == eid).astype(jnp.float32)
    counts = jnp.sum(onehot, axis=0, keepdims=True)
    tiles = jnp.floor((counts + (TM - 1)) / TM)
    r_e = lax.broadcasted_iota(jnp.int32, (NE, NE), 0)
    c_e = lax.broadcasted_iota(jnp.int32, (NE, NE), 1)
    lt_e = (r_e < c_e).astype(jnp.float32)
    tile_start = jnp.dot(tiles, lt_e, preferred_element_type=jnp.float32)
    row_start = tile_start * TM
    r_b = lax.broadcasted_iota(jnp.int32, (BLK, BLK), 0)
    c_b = lax.broadcasted_iota(jnp.int32, (BLK, BLK), 1)
    lt_b = (r_b > c_b).astype(jnp.float32)
    off = jnp.zeros((1, NE), jnp.float32)
    for b in range(S // BLK):
        ob = onehot[b * BLK:(b + 1) * BLK, :]
        ex = jnp.dot(lt_b, ob, preferred_element_type=jnp.float32) + off
        dest_b = jnp.sum(ob * (ex + row_start), axis=1, keepdims=True)
        dest_ref[b * BLK:(b + 1) * BLK, :] = dest_b.astype(jnp.int32)
        off = off + jnp.sum(ob, axis=0, keepdims=True)
    eye = (r_e == c_e).astype(jnp.float32)
    lo_ref[...] = jnp.sum(eye * (tile_start * TM), axis=1,
                          keepdims=True).astype(jnp.int32)
    nt_ref[...] = jnp.sum(eye * tiles, axis=1, keepdims=True).astype(jnp.int32)


NBUF = 3


def _expert_body(lo_ref, nt_ref, xs_ref, wg_hbm, wu_hbm, wd_hbm, o_ref,
                 wg_buf, wu_buf, wd_buf, sems):
    e = pl.program_id(0)

    def start_fetch(k):
        slot = lax.rem(k, NBUF)
        pltpu.make_async_copy(wg_hbm.at[k], wg_buf.at[slot],
                              sems.at[0, slot]).start()
        pltpu.make_async_copy(wu_hbm.at[k], wu_buf.at[slot],
                              sems.at[1, slot]).start()
        pltpu.make_async_copy(wd_hbm.at[k], wd_buf.at[slot],
                              sems.at[2, slot]).start()

    @pl.when(e == 0)
    def _prime():
        for k in range(NBUF):
            start_fetch(k)

    slot = lax.rem(e, NBUF)
    pltpu.make_async_copy(wg_hbm.at[e], wg_buf.at[slot], sems.at[0, slot]).wait()
    pltpu.make_async_copy(wu_hbm.at[e], wu_buf.at[slot], sems.at[1, slot]).wait()
    pltpu.make_async_copy(wd_hbm.at[e], wd_buf.at[slot], sems.at[2, slot]).wait()

    lo = lo_ref[e]
    nt = nt_ref[e]
    wg = wg_buf[slot]
    wu = wu_buf[slot]
    wd = wd_buf[slot]

    def tile_body(i, carry):
        r = pl.multiple_of(lo + i * TM, TM)
        xt = xs_ref[pl.ds(r, TM), :]
        g = jnp.dot(xt, wg, preferred_element_type=jnp.float32)
        u = jnp.dot(xt, wu, preferred_element_type=jnp.float32)
        a = g * jax.nn.sigmoid(g) * u
        o_ref[pl.ds(r, TM), :] = jnp.dot(a, wd,
                                         preferred_element_type=jnp.float32)
        return carry

    lax.fori_loop(0, nt, tile_body, 0)

    @pl.when(e + NBUF < NE)
    def _next():
        start_fetch(e + NBUF)


@functools.cache
def _sc_kernels():
    mesh = plsc.VectorSubcoreMesh(core_axis_name="c", subcore_axis_name="s")
    scratch = [
        pltpu.VMEM((TOK_W,), jnp.int32),
        pltpu.VMEM((TOK_W, H), jnp.float32),
        pltpu.SemaphoreType.DMA,
    ]

    @functools.partial(
        pl.kernel,
        out_type=jax.ShapeDtypeStruct((NROWS, H), jnp.float32),
        mesh=mesh,
        scratch_types=scratch,
    )
    def scatter_rows(x_hbm, dest_hbm, xs_hbm, idx_v, rows_v, sem):
        wid = lax.axis_index("s") * _NC + lax.axis_index("c")
        base = wid * TOK_W
        pltpu.sync_copy(dest_hbm.at[pl.ds(base, TOK_W)], idx_v)
        pltpu.sync_copy(x_hbm.at[pl.ds(base, TOK_W)], rows_v)
        pltpu.async_copy(rows_v, xs_hbm.at[idx_v], sem).wait()

    @functools.partial(
        pl.kernel,
        out_type=jax.ShapeDtypeStruct((S, H), jnp.float32),
        mesh=mesh,
        scratch_types=scratch,
    )
    def gather_rows(ys_hbm, dest_hbm, out_hbm, idx_v, rows_v, sem):
        wid = lax.axis_index("s") * _NC + lax.axis_index("c")
        base = wid * TOK_W
        pltpu.sync_copy(dest_hbm.at[pl.ds(base, TOK_W)], idx_v)
        pltpu.async_copy(ys_hbm.at[idx_v], rows_v, sem).wait()
        pltpu.sync_copy(rows_v, out_hbm.at[pl.ds(base, TOK_W)])

    return scatter_rows, gather_rows


def _route_call(x, wrt):
    return pl.pallas_call(
        _route_body,
        out_shape=[
            jax.ShapeDtypeStruct((S, 1), jnp.int32),
            jax.ShapeDtypeStruct((NE, 1), jnp.int32),
            jax.ShapeDtypeStruct((NE, 1), jnp.int32),
        ],
    )(x, wrt)


def _expert_call(lo, nt, xs, Wg, Wu, Wd):
    grid_spec = pltpu.PrefetchScalarGridSpec(
        num_scalar_prefetch=2,
        grid=(NE,),
        in_specs=[
            pl.BlockSpec((NROWS, H), lambda e, lo, nt: (0, 0)),
            pl.BlockSpec(memory_space=pltpu.MemorySpace.HBM),
            pl.BlockSpec(memory_space=pltpu.MemorySpace.HBM),
            pl.BlockSpec(memory_space=pltpu.MemorySpace.HBM),
        ],
        out_specs=pl.BlockSpec((NROWS, H), lambda e, lo, nt: (0, 0)),
        scratch_shapes=[
            pltpu.VMEM((NBUF, H, INNER), jnp.float32),
            pltpu.VMEM((NBUF, H, INNER), jnp.float32),
            pltpu.VMEM((NBUF, INNER, H), jnp.float32),
            pltpu.SemaphoreType.DMA((3, NBUF)),
        ],
    )
    return pl.pallas_call(
        _expert_body,
        grid_spec=grid_spec,
        out_shape=jax.ShapeDtypeStruct((NROWS, H), jnp.float32),
    )(lo, nt, xs, Wg, Wu, Wd)


def kernel(hidden_states, W_router, Wg, Wu, Wd):
    _, s, h = hidden_states.shape
    x = hidden_states.reshape(s, h)
    dest2, lo2, nt2 = _route_call(x, W_router.T)
    dest = dest2.reshape(S)
    lo = lo2.reshape(NE)
    nt = nt2.reshape(NE)
    scatter_rows, gather_rows = _sc_kernels()
    xs = scatter_rows(x, dest)
    ys = _expert_call(lo, nt, xs, Wg, Wu, Wd)
    out = gather_rows(ys, dest)
    return out.reshape(1, S, H)


# speedup vs baseline: 1.3558x; 1.3558x over previous
import functools

import jax
import jax.numpy as jnp
from jax import lax
from jax.experimental import pallas as pl
from jax.experimental.pallas import tpu as pltpu
from jax.experimental.pallas import tpu_sc as plsc

S = 2048
H = 768
INNER = 768
NE = 64
TM = 64
TILES = 96
NROWS = TILES * TM
BLK = 128

_NC, _NS = 2, 16
NW = _NC * _NS
TOK_W = S // NW


def _route_body(x_ref, wrt_ref, dest_ref, lo_ref, nt_ref):
    x = x_ref[...]
    logits = jnp.dot(x, wrt_ref[...], preferred_element_type=jnp.float32)
    iota_e = lax.broadcasted_iota(jnp.int32, (S, NE), 1)
    mx = jnp.max(logits, axis=1, keepdims=True)
    eid = jnp.min(jnp.where(logits == mx, iota_e, NE), axis=1, keepdims=True)
    onehot = (iota_e == eid).astype(jnp.float32)
    counts = jnp.sum(onehot, axis=0, keepdims=True)
    tiles = jnp.floor((counts + (TM - 1)) / TM)
    r_e = lax.broadcasted_iota(jnp.int32, (NE, NE), 0)
    c_e = lax.broadcasted_iota(jnp.int32, (NE, NE), 1)
    lt_e = (r_e < c_e).astype(jnp.float32)
    tile_start = jnp.dot(tiles, lt_e, preferred_element_type=jnp.float32)
    row_start = tile_start * TM
    r_b = lax.broadcasted_iota(jnp.int32, (BLK, BLK), 0)
    c_b = lax.broadcasted_iota(jnp.int32, (BLK, BLK), 1)
    lt_b = (r_b > c_b).astype(jnp.float32)
    off = jnp.zeros((1, NE), jnp.float32)
    for b in range(S // BLK):
        ob = onehot[b * BLK:(b + 1) * BLK, :]
        ex = jnp.dot(lt_b, ob, preferred_element_type=jnp.float32) + off
        dest_b = jnp.sum(ob * (ex + row_start), axis=1, keepdims=True)
        dest_ref[b * BLK:(b + 1) * BLK, :] = dest_b.astype(jnp.int32)
        off = off + jnp.sum(ob, axis=0, keepdims=True)
    eye = (r_e == c_e).astype(jnp.float32)
    lo_ref[...] = jnp.sum(eye * (tile_start * TM), axis=1,
                          keepdims=True).astype(jnp.int32)
    nt_ref[...] = jnp.sum(eye * tiles, axis=1, keepdims=True).astype(jnp.int32)


NBUF = 2


def _expert_body(lo_ref, nt_ref, xs_ref, wg_hbm, wu_hbm, wd_hbm, o_ref,
                 wg_buf, wu_buf, wd_buf, sems):
    e = pl.program_id(0)

    def start_fetch(k):
        slot = lax.rem(k, NBUF)
        pltpu.make_async_copy(wg_hbm.at[k], wg_buf.at[slot],
                              sems.at[0, slot]).start()
        pltpu.make_async_copy(wu_hbm.at[k], wu_buf.at[slot],
                              sems.at[1, slot]).start()
        pltpu.make_async_copy(wd_hbm.at[k], wd_buf.at[slot],
                              sems.at[2, slot]).start()

    @pl.when(e == 0)
    def _prime():
        for k in range(NBUF):
            start_fetch(k)

    slot = lax.rem(e, NBUF)
    pltpu.make_async_copy(wg_hbm.at[e], wg_buf.at[slot], sems.at[0, slot]).wait()
    pltpu.make_async_copy(wu_hbm.at[e], wu_buf.at[slot], sems.at[1, slot]).wait()
    pltpu.make_async_copy(wd_hbm.at[e], wd_buf.at[slot], sems.at[2, slot]).wait()

    lo = lo_ref[e]
    nt = nt_ref[e]
    wg = wg_buf[slot]
    wu = wu_buf[slot]
    wd = wd_buf[slot]

    def tile_body(i, carry):
        r = pl.multiple_of(lo + i * TM, TM)
        xt = xs_ref[pl.ds(r, TM), :]
        g = jnp.dot(xt, wg, preferred_element_type=jnp.float32)
        u = jnp.dot(xt, wu, preferred_element_type=jnp.float32)
        a = g * jax.nn.sigmoid(g) * u
        o_ref[pl.ds(r, TM), :] = jnp.dot(a, wd,
                                         preferred_element_type=jnp.float32)
        return carry

    lax.fori_loop(0, nt, tile_body, 0)

    @pl.when(e + NBUF < NE)
    def _next():
        start_fetch(e + NBUF)


@functools.cache
def _sc_kernels():
    mesh = plsc.VectorSubcoreMesh(core_axis_name="c", subcore_axis_name="s")
    scratch = [
        pltpu.VMEM((TOK_W,), jnp.int32),
        pltpu.VMEM((TOK_W, H), jnp.float32),
        pltpu.SemaphoreType.DMA,
    ]

    @functools.partial(
        pl.kernel,
        out_type=jax.ShapeDtypeStruct((NROWS, H), jnp.float32),
        mesh=mesh,
        scratch_types=scratch,
    )
    def scatter_rows(x_hbm, dest_hbm, xs_hbm, idx_v, rows_v, sem):
        wid = lax.axis_index("s") * _NC + lax.axis_index("c")
        base = wid * TOK_W
        pltpu.sync_copy(dest_hbm.at[pl.ds(base, TOK_W)], idx_v)
        pltpu.sync_copy(x_hbm.at[pl.ds(base, TOK_W)], rows_v)
        pltpu.async_copy(rows_v, xs_hbm.at[idx_v], sem).wait()

    @functools.partial(
        pl.kernel,
        out_type=jax.ShapeDtypeStruct((S, H), jnp.float32),
        mesh=mesh,
        scratch_types=scratch,
    )
    def gather_rows(ys_hbm, dest_hbm, out_hbm, idx_v, rows_v, sem):
        wid = lax.axis_index("s") * _NC + lax.axis_index("c")
        base = wid * TOK_W
        pltpu.sync_copy(dest_hbm.at[pl.ds(base, TOK_W)], idx_v)
        pltpu.async_copy(ys_hbm.at[idx_v], rows_v, sem).wait()
        pltpu.sync_copy(rows_v, out_hbm.at[pl.ds(base, TOK_W)])

    return scatter_rows, gather_rows


def _route_call(x, wrt):
    return pl.pallas_call(
        _route_body,
        out_shape=[
            jax.ShapeDtypeStruct((S, 1), jnp.int32),
            jax.ShapeDtypeStruct((NE, 1), jnp.int32),
            jax.ShapeDtypeStruct((NE, 1), jnp.int32),
        ],
    )(x, wrt)


def _expert_call(lo, nt, xs, Wg, Wu, Wd):
    grid_spec = pltpu.PrefetchScalarGridSpec(
        num_scalar_prefetch=2,
        grid=(NE,),
        in_specs=[
            pl.BlockSpec((NROWS, H), lambda e, lo, nt: (0, 0)),
            pl.BlockSpec(memory_space=pltpu.MemorySpace.HBM),
            pl.BlockSpec(memory_space=pltpu.MemorySpace.HBM),
            pl.BlockSpec(memory_space=pltpu.MemorySpace.HBM),
        ],
        out_specs=pl.BlockSpec((NROWS, H), lambda e, lo, nt: (0, 0)),
        scratch_shapes=[
            pltpu.VMEM((NBUF, H, INNER), jnp.float32),
            pltpu.VMEM((NBUF, H, INNER), jnp.float32),
            pltpu.VMEM((NBUF, INNER, H), jnp.float32),
            pltpu.SemaphoreType.DMA((3, NBUF)),
        ],
    )
    return pl.pallas_call(
        _expert_body,
        grid_spec=grid_spec,
        out_shape=jax.ShapeDtypeStruct((NROWS, H), jnp.float32),
    )(lo, nt, xs, Wg, Wu, Wd)


def kernel(hidden_states, W_router, Wg, Wu, Wd):
    _, s, h = hidden_states.shape
    x = hidden_states.reshape(s, h)
    dest2, lo2, nt2 = _route_call(x, W_router.T)
    dest = dest2.reshape(S)
    lo = lo2.reshape(NE)
    nt = nt2.reshape(NE)
    scatter_rows, gather_rows = _sc_kernels()
    xs = scatter_rows(x, dest)
    ys = _expert_call(lo, nt, xs, Wg, Wu, Wd)
    out = gather_rows(ys, dest)
    return out.reshape(1, S, H)


# speedup vs baseline: 1.4657x; 1.0810x over previous
import functools

import jax
import jax.numpy as jnp
from jax import lax
from jax.experimental import pallas as pl
from jax.experimental.pallas import tpu as pltpu
from jax.experimental.pallas import tpu_sc as plsc

S = 2048
H = 768
INNER = 768
NE = 64
TM = 32
TILES = 128
NROWS = TILES * TM
BLK = 128

_NC, _NS = 2, 16
NW = _NC * _NS
TOK_W = S // NW


def _route_body(x_ref, wrt_ref, dest_ref, lo_ref, nt_ref):
    x = x_ref[...]
    logits = jnp.dot(x, wrt_ref[...], preferred_element_type=jnp.float32)
    iota_e = lax.broadcasted_iota(jnp.int32, (S, NE), 1)
    mx = jnp.max(logits, axis=1, keepdims=True)
    eid = jnp.min(jnp.where(logits == mx, iota_e, NE), axis=1, keepdims=True)
    onehot = (iota_e == eid).astype(jnp.float32)
    counts = jnp.sum(onehot, axis=0, keepdims=True)
    tiles = jnp.floor((counts + (TM - 1)) / TM)
    r_e = lax.broadcasted_iota(jnp.int32, (NE, NE), 0)
    c_e = lax.broadcasted_iota(jnp.int32, (NE, NE), 1)
    lt_e = (r_e < c_e).astype(jnp.float32)
    tile_start = jnp.dot(tiles, lt_e, preferred_element_type=jnp.float32)
    row_start = tile_start * TM
    r_b = lax.broadcasted_iota(jnp.int32, (BLK, BLK), 0)
    c_b = lax.broadcasted_iota(jnp.int32, (BLK, BLK), 1)
    lt_b = (r_b > c_b).astype(jnp.float32)
    off = jnp.zeros((1, NE), jnp.float32)
    for b in range(S // BLK):
        ob = onehot[b * BLK:(b + 1) * BLK, :]
        ex = jnp.dot(lt_b, ob, preferred_element_type=jnp.float32) + off
        dest_b = jnp.sum(ob * (ex + row_start), axis=1, keepdims=True)
        dest_ref[b * BLK:(b + 1) * BLK, :] = dest_b.astype(jnp.int32)
        off = off + jnp.sum(ob, axis=0, keepdims=True)
    eye = (r_e == c_e).astype(jnp.float32)
    lo_ref[...] = jnp.sum(eye * (tile_start * TM), axis=1,
                          keepdims=True).astype(jnp.int32)
    nt_ref[...] = jnp.sum(eye * tiles, axis=1, keepdims=True).astype(jnp.int32)


NBUF = 3


def _expert_body(lo_ref, nt_ref, xs_ref, wg_hbm, wu_hbm, wd_hbm, o_ref,
                 wg_buf, wu_buf, wd_buf, sems):
    e = pl.program_id(0)

    def start_fetch(k):
        slot = lax.rem(k, NBUF)
        pltpu.make_async_copy(wg_hbm.at[k], wg_buf.at[slot],
                              sems.at[0, slot]).start()
        pltpu.make_async_copy(wu_hbm.at[k], wu_buf.at[slot],
                              sems.at[1, slot]).start()
        pltpu.make_async_copy(wd_hbm.at[k], wd_buf.at[slot],
                              sems.at[2, slot]).start()

    @pl.when(e == 0)
    def _prime():
        for k in range(NBUF):
            start_fetch(k)

    slot = lax.rem(e, NBUF)
    pltpu.make_async_copy(wg_hbm.at[e], wg_buf.at[slot], sems.at[0, slot]).wait()
    pltpu.make_async_copy(wu_hbm.at[e], wu_buf.at[slot], sems.at[1, slot]).wait()
    pltpu.make_async_copy(wd_hbm.at[e], wd_buf.at[slot], sems.at[2, slot]).wait()

    lo = lo_ref[e]
    nt = nt_ref[e]
    wg = wg_buf[slot]
    wu = wu_buf[slot]
    wd = wd_buf[slot]

    def tile_body(i, carry):
        r = pl.multiple_of(lo + i * TM, TM)
        xt = xs_ref[pl.ds(r, TM), :]
        g = jnp.dot(xt, wg, preferred_element_type=jnp.float32)
        u = jnp.dot(xt, wu, preferred_element_type=jnp.float32)
        a = g * jax.nn.sigmoid(g) * u
        o_ref[pl.ds(r, TM), :] = jnp.dot(a, wd,
                                         preferred_element_type=jnp.float32)
        return carry

    lax.fori_loop(0, nt, tile_body, 0)

    @pl.when(e + NBUF < NE)
    def _next():
        start_fetch(e + NBUF)


CH = 4
CW = TOK_W // CH


@functools.cache
def _sc_kernels():
    mesh = plsc.VectorSubcoreMesh(core_axis_name="c", subcore_axis_name="s")
    scratch = [
        pltpu.VMEM((CH, CW), jnp.int32),
        pltpu.VMEM((TOK_W, H), jnp.float32),
        pltpu.SemaphoreType.DMA,
        pltpu.SemaphoreType.DMA((CH,)),
        pltpu.SemaphoreType.DMA,
    ]

    @functools.partial(
        pl.kernel,
        out_type=jax.ShapeDtypeStruct((NROWS, H), jnp.float32),
        mesh=mesh,
        scratch_types=scratch,
    )
    def scatter_rows(x_hbm, dest_hbm, xs_hbm, idx_v, rows_v, sem_i, sems, sem_o):
        wid = lax.axis_index("s") * _NC + lax.axis_index("c")
        base = wid * TOK_W
        pltpu.make_async_copy(dest_hbm.at[wid], idx_v, sem_i).start()
        for c in range(CH):
            pltpu.make_async_copy(x_hbm.at[pl.ds(base + c * CW, CW)],
                                  rows_v.at[pl.ds(c * CW, CW)],
                                  sems.at[c]).start()
        pltpu.make_async_copy(dest_hbm.at[wid], idx_v, sem_i).wait()
        for c in range(CH):
            pltpu.make_async_copy(x_hbm.at[pl.ds(base + c * CW, CW)],
                                  rows_v.at[pl.ds(c * CW, CW)],
                                  sems.at[c]).wait()
            pltpu.make_async_copy(rows_v.at[pl.ds(c * CW, CW)],
                                  xs_hbm.at[idx_v.at[c]], sem_o).start()
        for c in range(CH):
            pltpu.make_async_copy(rows_v.at[pl.ds(c * CW, CW)],
                                  xs_hbm.at[idx_v.at[c]], sem_o).wait()

    @functools.partial(
        pl.kernel,
        out_type=jax.ShapeDtypeStruct((S, H), jnp.float32),
        mesh=mesh,
        scratch_types=scratch,
    )
    def gather_rows(ys_hbm, dest_hbm, out_hbm, idx_v, rows_v, sem_i, sems, sem_o):
        wid = lax.axis_index("s") * _NC + lax.axis_index("c")
        base = wid * TOK_W
        pltpu.make_async_copy(dest_hbm.at[wid], idx_v, sem_i).start()
        pltpu.make_async_copy(dest_hbm.at[wid], idx_v, sem_i).wait()
        for c in range(CH):
            pltpu.make_async_copy(ys_hbm.at[idx_v.at[c]],
                                  rows_v.at[pl.ds(c * CW, CW)],
                                  sems.at[c]).start()
        for c in range(CH):
            pltpu.make_async_copy(ys_hbm.at[idx_v.at[c]],
                                  rows_v.at[pl.ds(c * CW, CW)],
                                  sems.at[c]).wait()
            pltpu.make_async_copy(rows_v.at[pl.ds(c * CW, CW)],
                                  out_hbm.at[pl.ds(base + c * CW, CW)],
                                  sem_o).start()
        for c in range(CH):
            pltpu.make_async_copy(rows_v.at[pl.ds(c * CW, CW)],
                                  out_hbm.at[pl.ds(base + c * CW, CW)],
                                  sem_o).wait()

    return scatter_rows, gather_rows


def _route_call(x, wrt):
    return pl.pallas_call(
        _route_body,
        out_shape=[
            jax.ShapeDtypeStruct((S, 1), jnp.int32),
            jax.ShapeDtypeStruct((NE, 1), jnp.int32),
            jax.ShapeDtypeStruct((NE, 1), jnp.int32),
        ],
    )(x, wrt)


def _expert_call(lo, nt, xs, Wg, Wu, Wd):
    grid_spec = pltpu.PrefetchScalarGridSpec(
        num_scalar_prefetch=2,
        grid=(NE,),
        in_specs=[
            pl.BlockSpec((NROWS, H), lambda e, lo, nt: (0, 0)),
            pl.BlockSpec(memory_space=pltpu.MemorySpace.HBM),
            pl.BlockSpec(memory_space=pltpu.MemorySpace.HBM),
            pl.BlockSpec(memory_space=pltpu.MemorySpace.HBM),
        ],
        out_specs=pl.BlockSpec((NROWS, H), lambda e, lo, nt: (0, 0)),
        scratch_shapes=[
            pltpu.VMEM((NBUF, H, INNER), jnp.float32),
            pltpu.VMEM((NBUF, H, INNER), jnp.float32),
            pltpu.VMEM((NBUF, INNER, H), jnp.float32),
            pltpu.SemaphoreType.DMA((3, NBUF)),
        ],
    )
    return pl.pallas_call(
        _expert_body,
        grid_spec=grid_spec,
        out_shape=jax.ShapeDtypeStruct((NROWS, H), jnp.float32),
    )(lo, nt, xs, Wg, Wu, Wd)


def kernel(hidden_states, W_router, Wg, Wu, Wd):
    _, s, h = hidden_states.shape
    x = hidden_states.reshape(s, h)
    dest2, lo2, nt2 = _route_call(x, W_router.T)
    dest = dest2.reshape(NW, CH, CW)
    lo = lo2.reshape(NE)
    nt = nt2.reshape(NE)
    scatter_rows, gather_rows = _sc_kernels()
    xs = scatter_rows(x, dest)
    ys = _expert_call(lo, nt, xs, Wg, Wu, Wd)
    out = gather_rows(ys, dest)
    return out.reshape(1, S, H)


# speedup vs baseline: 1.4667x; 1.0007x over previous
import functools

import jax
import jax.numpy as jnp
from jax import lax
from jax.experimental import pallas as pl
from jax.experimental.pallas import tpu as pltpu
from jax.experimental.pallas import tpu_sc as plsc

S = 2048
H = 768
INNER = 768
NE = 64
TM = 32
TILES = 128
NROWS = TILES * TM
BLK = 128

_NC, _NS = 2, 16
NW = _NC * _NS
TOK_W = S // NW


def _route_body(x_ref, wrt_ref, dest_ref, lo_ref, nt_ref):
    x = x_ref[...]
    logits = jnp.dot(x, wrt_ref[...], preferred_element_type=jnp.float32)
    iota_e = lax.broadcasted_iota(jnp.int32, (S, NE), 1)
    mx = jnp.max(logits, axis=1, keepdims=True)
    eid = jnp.min(jnp.where(logits == mx, iota_e, NE), axis=1, keepdims=True)
    onehot = (iota_e == eid).astype(jnp.bfloat16)
    counts = jnp.sum(onehot.astype(jnp.float32), axis=0, keepdims=True)
    tiles = jnp.floor((counts + (TM - 1)) / TM)
    r_e = lax.broadcasted_iota(jnp.int32, (NE, NE), 0)
    c_e = lax.broadcasted_iota(jnp.int32, (NE, NE), 1)
    lt_e = (r_e < c_e).astype(jnp.bfloat16)
    tile_start = jnp.dot(tiles.astype(jnp.bfloat16), lt_e,
                         preferred_element_type=jnp.float32)
    row_start = tile_start * TM
    r_b = lax.broadcasted_iota(jnp.int32, (BLK, BLK), 0)
    c_b = lax.broadcasted_iota(jnp.int32, (BLK, BLK), 1)
    lt_b = (r_b > c_b).astype(jnp.bfloat16)
    off = jnp.zeros((1, NE), jnp.float32)
    for b in range(S // BLK):
        ob = onehot[b * BLK:(b + 1) * BLK, :]
        ex = jnp.dot(lt_b, ob, preferred_element_type=jnp.float32) + off
        obf = ob.astype(jnp.float32)
        dest_b = jnp.sum(obf * (ex + row_start), axis=1, keepdims=True)
        dest_ref[b * BLK:(b + 1) * BLK, :] = dest_b.astype(jnp.int32)
        off = off + jnp.sum(obf, axis=0, keepdims=True)
    eye = (r_e == c_e).astype(jnp.float32)
    lo_ref[...] = jnp.sum(eye * (tile_start * TM), axis=1,
                          keepdims=True).astype(jnp.int32)
    nt_ref[...] = jnp.sum(eye * tiles, axis=1, keepdims=True).astype(jnp.int32)


NBUF = 3


def _expert_body(lo_ref, nt_ref, xs_ref, wg_hbm, wu_hbm, wd_hbm, o_ref,
                 wg_buf, wu_buf, wd_buf, sems):
    e = pl.program_id(0)

    def start_fetch(k):
        slot = lax.rem(k, NBUF)
        pltpu.make_async_copy(wg_hbm.at[k], wg_buf.at[slot],
                              sems.at[0, slot]).start()
        pltpu.make_async_copy(wu_hbm.at[k], wu_buf.at[slot],
                              sems.at[1, slot]).start()
        pltpu.make_async_copy(wd_hbm.at[k], wd_buf.at[slot],
                              sems.at[2, slot]).start()

    @pl.when(e == 0)
    def _prime():
        for k in range(NBUF):
            start_fetch(k)

    slot = lax.rem(e, NBUF)
    pltpu.make_async_copy(wg_hbm.at[e], wg_buf.at[slot], sems.at[0, slot]).wait()
    pltpu.make_async_copy(wu_hbm.at[e], wu_buf.at[slot], sems.at[1, slot]).wait()
    pltpu.make_async_copy(wd_hbm.at[e], wd_buf.at[slot], sems.at[2, slot]).wait()

    lo = lo_ref[e]
    nt = nt_ref[e]
    wg = wg_buf[slot]
    wu = wu_buf[slot]
    wd = wd_buf[slot]

    def tile_body(i, carry):
        r = pl.multiple_of(lo + i * TM, TM)
        xt = xs_ref[pl.ds(r, TM), :]
        g = jnp.dot(xt, wg, preferred_element_type=jnp.float32)
        u = jnp.dot(xt, wu, preferred_element_type=jnp.float32)
        a = g * jax.nn.sigmoid(g) * u
        o_ref[pl.ds(r, TM), :] = jnp.dot(a, wd,
                                         preferred_element_type=jnp.float32)
        return carry

    lax.fori_loop(0, nt, tile_body, 0)

    @pl.when(e + NBUF < NE)
    def _next():
        start_fetch(e + NBUF)


CH = 4
CW = TOK_W // CH


@functools.cache
def _sc_kernels():
    mesh = plsc.VectorSubcoreMesh(core_axis_name="c", subcore_axis_name="s")
    scratch = [
        pltpu.VMEM((CH, CW), jnp.int32),
        pltpu.VMEM((TOK_W, H), jnp.float32),
        pltpu.SemaphoreType.DMA,
        pltpu.SemaphoreType.DMA((CH,)),
        pltpu.SemaphoreType.DMA,
    ]

    @functools.partial(
        pl.kernel,
        out_type=jax.ShapeDtypeStruct((NROWS, H), jnp.float32),
        mesh=mesh,
        scratch_types=scratch,
    )
    def scatter_rows(x_hbm, dest_hbm, xs_hbm, idx_v, rows_v, sem_i, sems, sem_o):
        wid = lax.axis_index("s") * _NC + lax.axis_index("c")
        base = wid * TOK_W
        pltpu.make_async_copy(dest_hbm.at[wid], idx_v, sem_i).start()
        for c in range(CH):
            pltpu.make_async_copy(x_hbm.at[pl.ds(base + c * CW, CW)],
                                  rows_v.at[pl.ds(c * CW, CW)],
                                  sems.at[c]).start()
        pltpu.make_async_copy(dest_hbm.at[wid], idx_v, sem_i).wait()
        for c in range(CH):
            pltpu.make_async_copy(x_hbm.at[pl.ds(base + c * CW, CW)],
                                  rows_v.at[pl.ds(c * CW, CW)],
                                  sems.at[c]).wait()
            pltpu.make_async_copy(rows_v.at[pl.ds(c * CW, CW)],
                                  xs_hbm.at[idx_v.at[c]], sem_o).start()
        for c in range(CH):
            pltpu.make_async_copy(rows_v.at[pl.ds(c * CW, CW)],
                                  xs_hbm.at[idx_v.at[c]], sem_o).wait()

    @functools.partial(
        pl.kernel,
        out_type=jax.ShapeDtypeStruct((S, H), jnp.float32),
        mesh=mesh,
        scratch_types=scratch,
    )
    def gather_rows(ys_hbm, dest_hbm, out_hbm, idx_v, rows_v, sem_i, sems, sem_o):
        wid = lax.axis_index("s") * _NC + lax.axis_index("c")
        base = wid * TOK_W
        pltpu.make_async_copy(dest_hbm.at[wid], idx_v, sem_i).start()
        pltpu.make_async_copy(dest_hbm.at[wid], idx_v, sem_i).wait()
        for c in range(CH):
            pltpu.make_async_copy(ys_hbm.at[idx_v.at[c]],
                                  rows_v.at[pl.ds(c * CW, CW)],
                                  sems.at[c]).start()
        for c in range(CH):
            pltpu.make_async_copy(ys_hbm.at[idx_v.at[c]],
                                  rows_v.at[pl.ds(c * CW, CW)],
                                  sems.at[c]).wait()
            pltpu.make_async_copy(rows_v.at[pl.ds(c * CW, CW)],
                                  out_hbm.at[pl.ds(base + c * CW, CW)],
                                  sem_o).start()
        for c in range(CH):
            pltpu.make_async_copy(rows_v.at[pl.ds(c * CW, CW)],
                                  out_hbm.at[pl.ds(base + c * CW, CW)],
                                  sem_o).wait()

    return scatter_rows, gather_rows


def _route_call(x, wrt):
    return pl.pallas_call(
        _route_body,
        out_shape=[
            jax.ShapeDtypeStruct((S, 1), jnp.int32),
            jax.ShapeDtypeStruct((NE, 1), jnp.int32),
            jax.ShapeDtypeStruct((NE, 1), jnp.int32),
        ],
    )(x, wrt)


def _expert_call(lo, nt, xs, Wg, Wu, Wd):
    grid_spec = pltpu.PrefetchScalarGridSpec(
        num_scalar_prefetch=2,
        grid=(NE,),
        in_specs=[
            pl.BlockSpec((NROWS, H), lambda e, lo, nt: (0, 0)),
            pl.BlockSpec(memory_space=pltpu.MemorySpace.HBM),
            pl.BlockSpec(memory_space=pltpu.MemorySpace.HBM),
            pl.BlockSpec(memory_space=pltpu.MemorySpace.HBM),
        ],
        out_specs=pl.BlockSpec((NROWS, H), lambda e, lo, nt: (0, 0)),
        scratch_shapes=[
            pltpu.VMEM((NBUF, H, INNER), jnp.float32),
            pltpu.VMEM((NBUF, H, INNER), jnp.float32),
            pltpu.VMEM((NBUF, INNER, H), jnp.float32),
            pltpu.SemaphoreType.DMA((3, NBUF)),
        ],
    )
    return pl.pallas_call(
        _expert_body,
        grid_spec=grid_spec,
        out_shape=jax.ShapeDtypeStruct((NROWS, H), jnp.float32),
    )(lo, nt, xs, Wg, Wu, Wd)


def kernel(hidden_states, W_router, Wg, Wu, Wd):
    _, s, h = hidden_states.shape
    x = hidden_states.reshape(s, h)
    dest2, lo2, nt2 = _route_call(x, W_router.T)
    dest = dest2.reshape(NW, CH, CW)
    lo = lo2.reshape(NE)
    nt = nt2.reshape(NE)
    scatter_rows, gather_rows = _sc_kernels()
    xs = scatter_rows(x, dest)
    ys = _expert_call(lo, nt, xs, Wg, Wu, Wd)
    out = gather_rows(ys, dest)
    return out.reshape(1, S, H)
